# baseline (speedup 1.0000x reference)
import functools

import jax
import jax.numpy as jnp
from jax import lax
from jax.experimental import pallas as pl
from jax.experimental.pallas import tpu as pltpu
from jax.experimental.pallas import tpu_sc as plsc

N = 10000
E = 320000
D = 128
H = 256
ACT = 20

NC = 2
NS = 16
NW = NC * NS

EW = E // NW
CH = 128
NCH = 80
SEC = 40
NBUF = 2
EPW = CH * NCH
EPAD = NW * EPW
NPAD = 10240
RPS = NPAD // NS

def _zero16():
    return jnp.zeros((16,), jnp.float32)


@functools.lru_cache(maxsize=1)
def _sc_kernels():
    mesh = plsc.VectorSubcoreMesh(core_axis_name="c", subcore_axis_name="s",
                                  num_cores=NC, num_subcores=NS)

    @functools.partial(
        pl.kernel,
        out_type=jax.ShapeDtypeStruct((2, NPAD), jnp.float32),
        mesh=mesh,
        scratch_types=[
            pltpu.VMEM((NCH, CH), jnp.int32),
            pltpu.VMEM((CH,), jnp.float32),
            pltpu.VMEM((RPS,), jnp.float32),
            pltpu.VMEM_SHARED((NPAD,), jnp.float32),
        ],
    )
    def deg_kernel(dst_hbm, out_hbm, idx_v, ones_v, zbuf_v, acc_sh):
        c = lax.axis_index("c")
        s = lax.axis_index("s")
        wid = s * NC + c

        for j in range(CH // 16):
            ones_v[pl.ds(j * 16, 16)] = jnp.full((16,), 1.0, jnp.float32)
        for j in range(RPS // 16):
            zbuf_v[pl.ds(j * 16, 16)] = _zero16()
        pltpu.sync_copy(zbuf_v, acc_sh.at[pl.ds(s * RPS, RPS)])
        plsc.subcore_barrier()

        pltpu.sync_copy(dst_hbm.at[pl.ds(wid * NCH, NCH)], idx_v)

        def body(ch):
            pltpu.sync_copy(ones_v, acc_sh.at[idx_v.at[ch]], add=True)

        pl.loop(0, NCH)(body)
        plsc.subcore_barrier()
        pltpu.sync_copy(acc_sh.at[pl.ds(s * RPS, RPS)],
                        out_hbm.at[c, pl.ds(s * RPS, RPS)])

    @functools.partial(
        pl.kernel,
        out_type=jax.ShapeDtypeStruct((2, NPAD, D), jnp.float32),
        mesh=mesh,
        scratch_types=[
            pltpu.VMEM((SEC * CH,), jnp.int32),
            pltpu.VMEM((SEC, CH), jnp.int32),
            [pltpu.VMEM((CH, D), jnp.float32)] * NBUF,
            [pltpu.SemaphoreType.DMA] * NBUF,
            [pltpu.SemaphoreType.DMA] * NBUF,
            pltpu.VMEM_SHARED((NPAD, D), jnp.float32),
        ],
    )
    def scatter_kernel(src_hbm, dst_hbm, g_hbm, out_hbm,
                       sidx_v, didx_v, bufs, gsems, ssems, acc_sh):
        c = lax.axis_index("c")
        s = lax.axis_index("s")
        wid = s * NC + c

        def zrow(r):
            for j in range(D // 16):
                bufs[0][r, pl.ds(j * 16, 16)] = _zero16()

        pl.loop(0, CH)(zrow)
        for t in range(RPS // CH):
            pltpu.sync_copy(bufs[0], acc_sh.at[pl.ds(s * RPS + t * CH, CH)])
        plsc.subcore_barrier()

        for sec in range(NCH // SEC):
            nreal = min(EW - sec * SEC * CH, SEC * CH)
            pltpu.sync_copy(
                src_hbm.at[pl.ds(wid * EW + sec * SEC * CH, nreal)],
                sidx_v.at[pl.ds(0, nreal)])
            for j in range((SEC * CH - nreal) // 16):
                sidx_v[pl.ds(nreal + j * 16, 16)] = (
                    N + j * 16 + lax.iota(jnp.int32, 16))
            pltpu.sync_copy(
                dst_hbm.at[pl.ds(wid * NCH + sec * SEC, SEC)], didx_v)
            pltpu.async_copy(
                g_hbm.at[sidx_v.at[pl.ds(0, CH)]], bufs[0], gsems[0])

            def body(l):
                for b in range(NBUF):
                    loc = l + b
                    nb = 1 - b

                    @pl.when(loc >= 1)
                    def _():
                        pltpu.make_async_copy(
                            bufs[nb], acc_sh.at[didx_v.at[loc - 1]],
                            ssems[nb]).wait()

                    @pl.when(loc + 1 < SEC)
                    def _():
                        pltpu.async_copy(
                            g_hbm.at[sidx_v.at[pl.ds((loc + 1) * CH, CH)]],
                            bufs[nb], gsems[nb])

                    pltpu.make_async_copy(
                        g_hbm.at[sidx_v.at[pl.ds(loc * CH, CH)]],
                        bufs[b], gsems[b]).wait()
                    pltpu.async_copy(
                        bufs[b], acc_sh.at[didx_v.at[loc]], ssems[b],
                        add=True)

            pl.loop(0, SEC, step=NBUF)(body)
            pltpu.make_async_copy(
                bufs[(SEC - 1) % NBUF], acc_sh.at[didx_v.at[SEC - 1]],
                ssems[(SEC - 1) % NBUF]).wait()
        plsc.subcore_barrier()
        pltpu.sync_copy(acc_sh.at[pl.ds(s * RPS, RPS)],
                        out_hbm.at[c, pl.ds(s * RPS, RPS)])

    return deg_kernel, scatter_kernel


_BR = 2048


def _matmul_body(x_ref, w_ref, h_ref):
    h_ref[...] = jnp.dot(x_ref[...], w_ref[...],
                         preferred_element_type=jnp.float32)


def _matmul_call(xp, W_gcn):
    return pl.pallas_call(
        _matmul_body,
        grid=(NPAD // _BR,),
        in_specs=[
            pl.BlockSpec((_BR, D), lambda i: (i, 0)),
            pl.BlockSpec((D, D), lambda i: (0, 0)),
        ],
        out_specs=pl.BlockSpec((_BR, D), lambda i: (i, 0)),
        out_shape=jax.ShapeDtypeStruct((NPAD, D), jnp.float32),
    )(xp, W_gcn)


def _scale_body(h_ref, degp_ref, g_ref, dinv_ref):
    ones2 = jnp.ones((2, 1), jnp.float32)
    degsum = lax.dot_general(degp_ref[...], ones2, (((0,), (0,)), ((), ())))
    dinv = lax.rsqrt(degsum + 1.0)
    g_ref[...] = h_ref[...] * dinv
    dinv_ref[...] = dinv


def _scale_call(h, degp):
    return pl.pallas_call(
        _scale_body,
        grid=(NPAD // _BR,),
        in_specs=[
            pl.BlockSpec((_BR, D), lambda i: (i, 0)),
            pl.BlockSpec((2, _BR), lambda i: (0, i)),
        ],
        out_specs=[
            pl.BlockSpec((_BR, D), lambda i: (i, 0)),
            pl.BlockSpec((_BR, 1), lambda i: (i, 0)),
        ],
        out_shape=[
            jax.ShapeDtypeStruct((NPAD, D), jnp.float32),
            jax.ShapeDtypeStruct((NPAD, 1), jnp.float32),
        ],
    )(h, degp)


_BR2 = 2000
_GB = _BR2 // ACT
_NB2 = N // _BR2


def _final_body(sp_ref, g_ref, x_ref, dinv_ref, bg_ref,
                w1_ref, b1_ref, w2_ref, b2_ref, w3_ref, b3_ref, out_ref):
    i = pl.program_id(0)
    ssum = sp_ref[0] + sp_ref[1] + g_ref[...]
    hres = jnp.maximum(ssum * dinv_ref[...] + bg_ref[...], 0.0) + x_ref[...]
    a1 = jnp.maximum(
        jnp.dot(hres, w1_ref[...], preferred_element_type=jnp.float32)
        + b1_ref[...], 0.0)
    a2 = jnp.maximum(
        jnp.dot(a1, w2_ref[...], preferred_element_type=jnp.float32)
        + b2_ref[...], 0.0)
    gs = jnp.sum(a2.reshape(_GB, ACT, H), axis=1)
    res = jnp.dot(gs, w3_ref[...], preferred_element_type=jnp.float32) \
        + b3_ref[...]
    out_ref[pl.ds(i * _GB, _GB), :] = res


def _final_call(sp, g, x, dinvc, bg, W1, b1, W2, b2, W3, b3):
    return pl.pallas_call(
        _final_body,
        grid=(_NB2,),
        in_specs=[
            pl.BlockSpec((2, _BR2, D), lambda i: (0, i, 0)),
            pl.BlockSpec((_BR2, D), lambda i: (i, 0)),
            pl.BlockSpec((_BR2, D), lambda i: (i, 0)),
            pl.BlockSpec((_BR2, 1), lambda i: (i, 0)),
            pl.BlockSpec((1, D), lambda i: (0, 0)),
            pl.BlockSpec((D, H), lambda i: (0, 0)),
            pl.BlockSpec((1, H), lambda i: (0, 0)),
            pl.BlockSpec((H, H), lambda i: (0, 0)),
            pl.BlockSpec((1, H), lambda i: (0, 0)),
            pl.BlockSpec((H, 1), lambda i: (0, 0)),
            pl.BlockSpec((1, 1), lambda i: (0, 0)),
        ],
        out_specs=pl.BlockSpec((N // ACT, 1), lambda i: (0, 0)),
        out_shape=jax.ShapeDtypeStruct((N // ACT, 1), jnp.float32),
    )(sp, g, x, dinvc, bg, W1, b1, W2, b2, W3, b3)


def kernel(x, edge_index, W_gcn, b_gcn, W1, b1, W2, b2, W3, b3):
    ppt = EPW - EW
    padv = N + (jnp.arange(ppt, dtype=jnp.int32) % (NPAD - N))
    padb = jnp.broadcast_to(padv, (NW, ppt))
    dst2d = jnp.concatenate([edge_index[1].reshape(NW, EW), padb],
                            axis=1).reshape(NW * NCH, CH)
    xp = jnp.pad(x, ((0, NPAD - N), (0, 0)))

    deg_kernel, scatter_kernel = _sc_kernels()
    degp = deg_kernel(dst2d)
    h = _matmul_call(xp, W_gcn)
    g, dinvc = _scale_call(h, degp)
    sp = scatter_kernel(edge_index[0], dst2d, g)
    out = _final_call(sp, g, x, dinvc,
                      b_gcn.reshape(1, D), W1, b1.reshape(1, H),
                      W2, b2.reshape(1, H), W3, b3.reshape(1, 1))
    return out


# speedup vs baseline: 1.0786x; 1.0786x over previous
import functools

import jax
import jax.numpy as jnp
from jax import lax
from jax.experimental import pallas as pl
from jax.experimental.pallas import tpu as pltpu
from jax.experimental.pallas import tpu_sc as plsc

N = 10000
E = 320000
D = 128
H = 256
ACT = 20

NC = 2
NS = 16
NW = NC * NS

EW = E // NW
CH = 128
NCH = 80
SEC = 40
NBUF = 2
EPW = CH * NCH
EPAD = NW * EPW
NPAD = 10240
RPS = NPAD // NS

def _zero16():
    return jnp.zeros((16,), jnp.float32)


@functools.lru_cache(maxsize=1)
def _sc_kernels():
    mesh = plsc.VectorSubcoreMesh(core_axis_name="c", subcore_axis_name="s",
                                  num_cores=NC, num_subcores=NS)

    @functools.partial(
        pl.kernel,
        out_type=jax.ShapeDtypeStruct((2, NPAD), jnp.float32),
        mesh=mesh,
        scratch_types=[
            pltpu.VMEM((NCH, CH), jnp.int32),
            pltpu.VMEM((CH,), jnp.float32),
            pltpu.VMEM((RPS,), jnp.float32),
            pltpu.VMEM_SHARED((NPAD,), jnp.float32),
        ],
    )
    def deg_kernel(edge_hbm, out_hbm, idx_v, ones_v, zbuf_v, acc_sh):
        c = lax.axis_index("c")
        s = lax.axis_index("s")
        wid = s * NC + c

        for j in range(CH // 16):
            ones_v[pl.ds(j * 16, 16)] = jnp.full((16,), 1.0, jnp.float32)
        for j in range(RPS // 16):
            zbuf_v[pl.ds(j * 16, 16)] = _zero16()
        pltpu.sync_copy(zbuf_v, acc_sh.at[pl.ds(s * RPS, RPS)])
        plsc.subcore_barrier()

        pltpu.sync_copy(edge_hbm.at[1, pl.ds(wid * NCH, NCH)], idx_v)

        def body(ch):
            pltpu.sync_copy(ones_v, acc_sh.at[idx_v.at[ch]], add=True)

        pl.loop(0, NCH)(body)
        plsc.subcore_barrier()
        pltpu.sync_copy(acc_sh.at[pl.ds(s * RPS, RPS)],
                        out_hbm.at[c, pl.ds(s * RPS, RPS)])

    @functools.partial(
        pl.kernel,
        out_type=jax.ShapeDtypeStruct((2, NPAD, D), jnp.float32),
        mesh=mesh,
        scratch_types=[
            pltpu.VMEM((SEC, CH), jnp.int32),
            pltpu.VMEM((SEC, CH), jnp.int32),
            [pltpu.VMEM((CH, D), jnp.float32)] * NBUF,
            [pltpu.SemaphoreType.DMA] * NBUF,
            [pltpu.SemaphoreType.DMA] * NBUF,
            pltpu.VMEM_SHARED((NPAD, D), jnp.float32),
        ],
    )
    def scatter_kernel(edge_hbm, g_hbm, out_hbm,
                       sidx_v, didx_v, bufs, gsems, ssems, acc_sh):
        c = lax.axis_index("c")
        s = lax.axis_index("s")
        wid = s * NC + c

        def zrow(r):
            for j in range(D // 16):
                bufs[0][r, pl.ds(j * 16, 16)] = _zero16()

        pl.loop(0, CH)(zrow)
        for t in range(RPS // CH):
            pltpu.sync_copy(bufs[0], acc_sh.at[pl.ds(s * RPS + t * CH, CH)])
        plsc.subcore_barrier()

        for sec in range(NCH // SEC):
            base = wid * NCH + sec * SEC
            pltpu.sync_copy(edge_hbm.at[0, pl.ds(base, SEC)], sidx_v)
            pltpu.sync_copy(edge_hbm.at[1, pl.ds(base, SEC)], didx_v)
            pltpu.async_copy(g_hbm.at[sidx_v.at[0]], bufs[0], gsems[0])

            def body(l):
                for b in range(NBUF):
                    loc = l + b
                    nb = 1 - b

                    @pl.when(loc >= 1)
                    def _():
                        pltpu.make_async_copy(
                            bufs[nb], acc_sh.at[didx_v.at[loc - 1]],
                            ssems[nb]).wait()

                    @pl.when(loc + 1 < SEC)
                    def _():
                        pltpu.async_copy(
                            g_hbm.at[sidx_v.at[loc + 1]], bufs[nb], gsems[nb])

                    pltpu.make_async_copy(
                        g_hbm.at[sidx_v.at[loc]], bufs[b], gsems[b]).wait()
                    pltpu.async_copy(
                        bufs[b], acc_sh.at[didx_v.at[loc]], ssems[b],
                        add=True)

            pl.loop(0, SEC, step=NBUF)(body)
            pltpu.make_async_copy(
                bufs[(SEC - 1) % NBUF], acc_sh.at[didx_v.at[SEC - 1]],
                ssems[(SEC - 1) % NBUF]).wait()
        plsc.subcore_barrier()
        pltpu.sync_copy(acc_sh.at[pl.ds(s * RPS, RPS)],
                        out_hbm.at[c, pl.ds(s * RPS, RPS)])

    return deg_kernel, scatter_kernel


_BR = 2048


def _matmul_body(x_ref, w_ref, h_ref):
    h_ref[...] = jnp.dot(x_ref[...], w_ref[...],
                         preferred_element_type=jnp.float32)


def _matmul_call(xp, W_gcn):
    return pl.pallas_call(
        _matmul_body,
        grid=(NPAD // _BR,),
        in_specs=[
            pl.BlockSpec((_BR, D), lambda i: (i, 0)),
            pl.BlockSpec((D, D), lambda i: (0, 0)),
        ],
        out_specs=pl.BlockSpec((_BR, D), lambda i: (i, 0)),
        out_shape=jax.ShapeDtypeStruct((NPAD, D), jnp.float32),
    )(xp, W_gcn)


def _scale_body(h_ref, degp_ref, g_ref, dinv_ref):
    ones2 = jnp.ones((2, 1), jnp.float32)
    degsum = lax.dot_general(degp_ref[...], ones2, (((0,), (0,)), ((), ())))
    dinv = lax.rsqrt(degsum + 1.0)
    g_ref[...] = h_ref[...] * dinv
    dinv_ref[...] = dinv


def _scale_call(h, degp):
    return pl.pallas_call(
        _scale_body,
        grid=(NPAD // _BR,),
        in_specs=[
            pl.BlockSpec((_BR, D), lambda i: (i, 0)),
            pl.BlockSpec((2, _BR), lambda i: (0, i)),
        ],
        out_specs=[
            pl.BlockSpec((_BR, D), lambda i: (i, 0)),
            pl.BlockSpec((_BR, 1), lambda i: (i, 0)),
        ],
        out_shape=[
            jax.ShapeDtypeStruct((NPAD, D), jnp.float32),
            jax.ShapeDtypeStruct((NPAD, 1), jnp.float32),
        ],
    )(h, degp)


_BR2 = 2000
_GB = _BR2 // ACT
_NB2 = N // _BR2


def _final_body(sp_ref, g_ref, x_ref, dinv_ref, bg_ref,
                w1_ref, b1_ref, w2_ref, b2_ref, w3_ref, b3_ref, out_ref):
    i = pl.program_id(0)
    ssum = sp_ref[0] + sp_ref[1] + g_ref[...]
    hres = jnp.maximum(ssum * dinv_ref[...] + bg_ref[...], 0.0) + x_ref[...]
    a1 = jnp.maximum(
        jnp.dot(hres, w1_ref[...], preferred_element_type=jnp.float32)
        + b1_ref[...], 0.0)
    a2 = jnp.maximum(
        jnp.dot(a1, w2_ref[...], preferred_element_type=jnp.float32)
        + b2_ref[...], 0.0)
    gs = jnp.sum(a2.reshape(_GB, ACT, H), axis=1)
    res = jnp.dot(gs, w3_ref[...], preferred_element_type=jnp.float32) \
        + b3_ref[...]
    out_ref[pl.ds(i * _GB, _GB), :] = res


def _final_call(sp, g, x, dinvc, bg, W1, b1, W2, b2, W3, b3):
    return pl.pallas_call(
        _final_body,
        grid=(_NB2,),
        in_specs=[
            pl.BlockSpec((2, _BR2, D), lambda i: (0, i, 0)),
            pl.BlockSpec((_BR2, D), lambda i: (i, 0)),
            pl.BlockSpec((_BR2, D), lambda i: (i, 0)),
            pl.BlockSpec((_BR2, 1), lambda i: (i, 0)),
            pl.BlockSpec((1, D), lambda i: (0, 0)),
            pl.BlockSpec((D, H), lambda i: (0, 0)),
            pl.BlockSpec((1, H), lambda i: (0, 0)),
            pl.BlockSpec((H, H), lambda i: (0, 0)),
            pl.BlockSpec((1, H), lambda i: (0, 0)),
            pl.BlockSpec((H, 1), lambda i: (0, 0)),
            pl.BlockSpec((1, 1), lambda i: (0, 0)),
        ],
        out_specs=pl.BlockSpec((N // ACT, 1), lambda i: (0, 0)),
        out_shape=jax.ShapeDtypeStruct((N // ACT, 1), jnp.float32),
    )(sp, g, x, dinvc, bg, W1, b1, W2, b2, W3, b3)


def kernel(x, edge_index, W_gcn, b_gcn, W1, b1, W2, b2, W3, b3):
    ppt = EPW - EW
    padv = N + (jnp.arange(ppt, dtype=jnp.int32) % (NPAD - N))
    padb = jnp.broadcast_to(padv, (2, NW, ppt))
    e2d = jnp.concatenate([edge_index.reshape(2, NW, EW), padb],
                          axis=2).reshape(2, NW * NCH, CH)
    xp = jnp.pad(x, ((0, NPAD - N), (0, 0)))

    deg_kernel, scatter_kernel = _sc_kernels()
    degp = deg_kernel(e2d)
    h = _matmul_call(xp, W_gcn)
    g, dinvc = _scale_call(h, degp)
    sp = scatter_kernel(e2d, g)
    out = _final_call(sp, g, x, dinvc,
                      b_gcn.reshape(1, D), W1, b1.reshape(1, H),
                      W2, b2.reshape(1, H), W3, b3.reshape(1, 1))
    return out
